# Initial kernel scaffold; baseline (speedup 1.0000x reference)
import jax
import jax.numpy as jnp
from jax.experimental import pallas as pl


def kernel(x, edge_index, edge_attr, u, batch, W1, b1, W2, b2):
    raise NotImplementedError("write your pallas kernel here")


# speedup vs baseline: 1.0045x; 1.0045x over previous
import functools

import jax
import jax.numpy as jnp
from jax.experimental import pallas as pl
from jax.experimental.pallas import tpu as pltpu

N = 100000
E = 3200000
D = 128
DE = 16
G = 8
H = 256
IN_DIM = D + 3 * DE + G * 0 + 16

_BLK = 2000


def _mlp_body(x_ref, s_ref, m_ref, mn_ref, b_ref, u_ref, w1_ref, b1_ref,
              w2_ref, b2_ref, o_ref):
    x = x_ref[...]
    oh = (b_ref[...] == jax.lax.broadcasted_iota(jnp.int32, (_BLK, G), 1))
    ub = jnp.dot(oh.astype(jnp.float32), u_ref[...],
                 preferred_element_type=jnp.float32)
    cat = jnp.concatenate([x, s_ref[...], m_ref[...], mn_ref[...], ub], axis=1)
    h = jnp.dot(cat, w1_ref[...], preferred_element_type=jnp.float32)
    h = jnp.maximum(h + b1_ref[...], 0.0)
    o = jnp.dot(h, w2_ref[...], preferred_element_type=jnp.float32)
    o_ref[...] = o + b2_ref[...] + x


def _mlp(x, out1, out2, out3, batch2d, u, W1, b1, W2, b2):
    nblk = N // _BLK
    rep = lambda i: (0, 0)
    return pl.pallas_call(
        _mlp_body,
        grid=(nblk,),
        in_specs=[
            pl.BlockSpec((_BLK, D), lambda i: (i, 0)),
            pl.BlockSpec((_BLK, DE), lambda i: (i, 0)),
            pl.BlockSpec((_BLK, DE), lambda i: (i, 0)),
            pl.BlockSpec((_BLK, DE), lambda i: (i, 0)),
            pl.BlockSpec((_BLK, 1), lambda i: (i, 0)),
            pl.BlockSpec((G, 16), rep),
            pl.BlockSpec((IN_DIM, H), rep),
            pl.BlockSpec((1, H), rep),
            pl.BlockSpec((H, D), rep),
            pl.BlockSpec((1, D), rep),
        ],
        out_specs=pl.BlockSpec((_BLK, D), lambda i: (i, 0)),
        out_shape=jax.ShapeDtypeStruct((N, D), jnp.float32),
    )(x, out1, out2, out3, batch2d, u, W1, b1, W2, b2)


def kernel(x, edge_index, edge_attr, u, batch, W1, b1, W2, b2):
    col = edge_index[1]
    out1 = jax.ops.segment_sum(edge_attr, col, num_segments=N)
    cnt = jax.ops.segment_sum(jnp.ones((E, 1), jnp.float32), col, num_segments=N)
    maxv = jax.ops.segment_max(edge_attr, col, num_segments=N)
    out2 = jnp.where(cnt > 0, maxv, 0.0)
    out3 = out1 / jnp.maximum(cnt, 1.0)
    return _mlp(x, out1, out2, out3, batch.reshape(N, 1), u,
                W1, b1.reshape(1, H), W2, b2.reshape(1, D))


# speedup vs baseline: 1.7542x; 1.7463x over previous
import functools

import jax
import jax.numpy as jnp
from jax import lax
from jax.experimental import pallas as pl
from jax.experimental.pallas import tpu as pltpu
from jax.experimental.pallas import tpu_sc as plsc

N = 100000
E = 3200000
D = 128
DE = 16
G = 8
H = 256
IN_DIM = D + 3 * DE + 16

_NTILES = 32
_EPT = E // _NTILES
_CH = 1000
_NPAD = 100096
_NPT = _NPAD // 16
_ZCH = 368


def _sc_sumcnt_body(col_hbm, attr_hbm, ones_hbm, zrow_hbm, z1_hbm,
                    osum_hbm, ocnt_hbm,
                    acc_s, acc_c, attr_v, col_v, ones_v):
    cid = lax.axis_index("c")
    wid = lax.axis_index("s") * 2 + cid
    pltpu.sync_copy(ones_hbm, ones_v)
    row0 = wid // 2 * _NPT
    @pl.loop(0, _NPT, step=_ZCH)
    def _(r):
        pltpu.sync_copy(zrow_hbm, acc_s.at[pl.ds(row0 + r, _ZCH)])
        pltpu.sync_copy(z1_hbm, acc_c.at[pl.ds(row0 + r, _ZCH)])
    plsc.subcore_barrier()
    base_e = wid * _EPT
    @pl.loop(0, _EPT, step=_CH)
    def _(e0):
        pltpu.sync_copy(col_hbm.at[pl.ds(base_e + e0, _CH)], col_v)
        pltpu.sync_copy(attr_hbm.at[pl.ds(base_e + e0, _CH)], attr_v)
        pltpu.sync_copy(attr_v, acc_s.at[col_v], add=True)
        pltpu.sync_copy(ones_v, acc_c.at[col_v], add=True)
    plsc.subcore_barrier()
    pltpu.sync_copy(acc_s.at[pl.ds(row0, _NPT)],
                    osum_hbm.at[cid].at[pl.ds(row0, _NPT)])
    pltpu.sync_copy(acc_c.at[pl.ds(row0, _NPT)],
                    ocnt_hbm.at[cid].at[pl.ds(row0, _NPT)])


def _sc_sumcnt(col, attr):
    mesh = plsc.VectorSubcoreMesh(core_axis_name="c", subcore_axis_name="s")
    run = pl.kernel(
        _sc_sumcnt_body,
        mesh=mesh,
        compiler_params=pltpu.CompilerParams(use_tc_tiling_on_sc=False),
        out_type=(jax.ShapeDtypeStruct((2, _NPAD, DE), jnp.float32),
                  jax.ShapeDtypeStruct((2, _NPAD), jnp.float32)),
        scratch_types=[
            pltpu.VMEM_SHARED((_NPAD, DE), jnp.float32),
            pltpu.VMEM_SHARED((_NPAD,), jnp.float32),
            pltpu.VMEM((_CH, DE), jnp.float32),
            pltpu.VMEM((_CH,), jnp.int32),
            pltpu.VMEM((_CH,), jnp.float32),
        ],
    )
    return run(col, attr, jnp.ones((_CH,), jnp.float32),
               jnp.zeros((_ZCH, DE), jnp.float32),
               jnp.zeros((_ZCH,), jnp.float32))


_BLK = 2000


def _mlp_body(x_ref, s_ref, c_ref, m_ref, b_ref, u_ref, w1_ref, b1_ref,
              w2_ref, b2_ref, o_ref):
    x = x_ref[...]
    s = s_ref[0] + s_ref[1]
    c = c_ref[0] + c_ref[1]
    m = jnp.where(c > 0, m_ref[...], 0.0)
    mn = s / jnp.maximum(c, 1.0)
    oh = (b_ref[...] == jax.lax.broadcasted_iota(jnp.int32, (_BLK, G), 1))
    ub = jnp.dot(oh.astype(jnp.float32), u_ref[...],
                 preferred_element_type=jnp.float32)
    cat = jnp.concatenate([x, s, m, mn, ub], axis=1)
    h = jnp.dot(cat, w1_ref[...], preferred_element_type=jnp.float32)
    h = jnp.maximum(h + b1_ref[...], 0.0)
    o = jnp.dot(h, w2_ref[...], preferred_element_type=jnp.float32)
    o_ref[...] = o + b2_ref[...] + x


def _mlp(x, sum2, cnt2, maxraw, batch2d, u, W1, b1, W2, b2):
    nblk = N // _BLK
    rep = lambda i: (0, 0)
    return pl.pallas_call(
        _mlp_body,
        grid=(nblk,),
        in_specs=[
            pl.BlockSpec((_BLK, D), lambda i: (i, 0)),
            pl.BlockSpec((2, _BLK, DE), lambda i: (0, i, 0)),
            pl.BlockSpec((2, _BLK, 1), lambda i: (0, i, 0)),
            pl.BlockSpec((_BLK, DE), lambda i: (i, 0)),
            pl.BlockSpec((_BLK, 1), lambda i: (i, 0)),
            pl.BlockSpec((G, 16), rep),
            pl.BlockSpec((IN_DIM, H), rep),
            pl.BlockSpec((1, H), rep),
            pl.BlockSpec((H, D), rep),
            pl.BlockSpec((1, D), rep),
        ],
        out_specs=pl.BlockSpec((_BLK, D), lambda i: (i, 0)),
        out_shape=jax.ShapeDtypeStruct((N, D), jnp.float32),
    )(x, sum2, cnt2, maxraw, batch2d, u, W1, b1, W2, b2)


def kernel(x, edge_index, edge_attr, u, batch, W1, b1, W2, b2):
    col = edge_index[1]
    sum2, cnt2 = _sc_sumcnt(col, edge_attr)
    cnt2 = cnt2.reshape(2, _NPAD, 1)
    maxraw = jax.ops.segment_max(edge_attr, col, num_segments=N)
    return _mlp(x, sum2, cnt2, maxraw,
                batch.reshape(N, 1), u, W1, b1.reshape(1, H), W2,
                b2.reshape(1, D))


# speedup vs baseline: 6.4541x; 3.6791x over previous
import dataclasses
import functools

import jax
import jax.numpy as jnp
from jax import lax
from jax.experimental import pallas as pl
from jax.experimental.pallas import tpu as pltpu
from jax.experimental.pallas import tpu_sc as plsc

N = 100000
E = 3200000
D = 128
DE = 16
G = 8
H = 256
IN_DIM = D + 3 * DE + 16

_NTILES = 32
_EPT = E // _NTILES
_CH = 1000
_NPAD = 100096
_NPT = _NPAD // 16
_ZCH = 368
_CH3 = 800
_NPT32 = _NPAD // 32
_WIN = 1000
_EPAD = E + _WIN + 8

_SC_PARAMS = pltpu.CompilerParams(use_tc_tiling_on_sc=False)
_SC_PARAMS_NL = (
    dataclasses.replace(_SC_PARAMS, needs_layout_passes=False)
    if "needs_layout_passes" in pltpu.CompilerParams.__dataclass_fields__
    else _SC_PARAMS)


def _sc_sumcnt_body(col_hbm, attr_hbm, ones_hbm, zrow_hbm, z1_hbm,
                    osum_hbm, ocnt_hbm,
                    acc_s, acc_c, attr_v, col_v, ones_v):
    cid = lax.axis_index("c")
    wid = lax.axis_index("s") * 2 + cid
    pltpu.sync_copy(ones_hbm, ones_v)
    row0 = wid // 2 * _NPT
    @pl.loop(0, _NPT, step=_ZCH)
    def _(r):
        pltpu.sync_copy(zrow_hbm, acc_s.at[pl.ds(row0 + r, _ZCH)])
        pltpu.sync_copy(z1_hbm, acc_c.at[pl.ds(row0 + r, _ZCH)])
    plsc.subcore_barrier()
    base_e = wid * _EPT
    @pl.loop(0, _EPT, step=_CH)
    def _(e0):
        pltpu.sync_copy(col_hbm.at[pl.ds(base_e + e0, _CH)], col_v)
        pltpu.sync_copy(attr_hbm.at[pl.ds(base_e + e0, _CH)], attr_v)
        pltpu.sync_copy(attr_v, acc_s.at[col_v], add=True)
        pltpu.sync_copy(ones_v, acc_c.at[col_v], add=True)
    plsc.subcore_barrier()
    pltpu.sync_copy(acc_s.at[pl.ds(row0, _NPT)],
                    osum_hbm.at[cid].at[pl.ds(row0, _NPT)])
    pltpu.sync_copy(acc_c.at[pl.ds(row0, _NPT)],
                    ocnt_hbm.at[cid].at[pl.ds(row0, _NPT)])


def _sc_sumcnt(col, attr):
    mesh = plsc.VectorSubcoreMesh(core_axis_name="c", subcore_axis_name="s")
    run = pl.kernel(
        _sc_sumcnt_body,
        mesh=mesh,
        compiler_params=_SC_PARAMS,
        out_type=(jax.ShapeDtypeStruct((2, _NPAD, DE), jnp.float32),
                  jax.ShapeDtypeStruct((2, _NPAD), jnp.float32)),
        scratch_types=[
            pltpu.VMEM_SHARED((_NPAD, DE), jnp.float32),
            pltpu.VMEM_SHARED((_NPAD,), jnp.float32),
            pltpu.VMEM((_CH, DE), jnp.float32),
            pltpu.VMEM((_CH,), jnp.int32),
            pltpu.VMEM((_CH,), jnp.float32),
        ],
    )
    return run(col, attr, jnp.ones((_CH,), jnp.float32),
               jnp.zeros((_ZCH, DE), jnp.float32),
               jnp.zeros((_ZCH,), jnp.float32))


def _lane_take(x, idx):
    dnums = lax.GatherDimensionNumbers(
        offset_dims=(), collapsed_slice_dims=(0,), start_index_map=(0,))
    return lax.gather(x, idx[:, None], dnums, (1,),
                      mode=lax.GatherScatterMode.PROMISE_IN_BOUNDS)


def _vec_ranks(cv):
    lane = lax.iota(jnp.int32, 16)
    sk, sv = plsc.sort_key_val(cv, lane)
    prev = _lane_take(sk, jnp.maximum(lane - 1, 0))
    bnd = jnp.logical_or(lane == 0, sk != prev)
    start = plsc.cummax(jnp.where(bnd, lane, 0))
    rank = lane - start
    nxt = _lane_take(sk, jnp.minimum(lane + 1, 15))
    last = jnp.logical_or(lane == 15, sk != nxt)
    return sk, sv, rank, last


def _sc_hist_body(col_hbm, zi_hbm, hist_hbm, hist_v, col_v):
    wid = lax.axis_index("s") * 2 + lax.axis_index("c")
    @pl.loop(0, _NPAD, step=_NPT)
    def _(r):
        pltpu.sync_copy(zi_hbm, hist_v.at[pl.ds(r, _NPT)])
    base_e = wid * _EPT
    @pl.loop(0, _EPT, step=_CH3)
    def _(e0):
        pltpu.sync_copy(col_hbm.at[pl.ds(base_e + e0, _CH3)], col_v)
        @pl.loop(0, _CH3, step=16)
        def _(v):
            cv = col_v[pl.ds(v, 16)]
            sk, _sv, rank, last = _vec_ranks(cv)
            plsc.addupdate_scatter(hist_v, [sk], rank + 1, mask=last)
    pltpu.sync_copy(hist_v, hist_hbm.at[wid])


def _sc_hist(col):
    mesh = plsc.VectorSubcoreMesh(core_axis_name="c", subcore_axis_name="s")
    run = pl.kernel(
        _sc_hist_body,
        mesh=mesh,
        compiler_params=_SC_PARAMS_NL,
        out_type=jax.ShapeDtypeStruct((_NTILES, _NPAD), jnp.int32),
        scratch_types=[
            pltpu.VMEM((_NPAD,), jnp.int32),
            pltpu.VMEM((_CH3,), jnp.int32),
        ],
    )
    return run(col, jnp.zeros((_NPT,), jnp.int32))


_BC = 4352


def _offsets_body(h_ref, bases_ref, csr_ref, cnt_ref, carry_ref):
    @pl.when(pl.program_id(0) == 0)
    def _():
        carry_ref[0, 0] = 0
    h = h_ref[...]
    tot = jnp.sum(h, axis=0, keepdims=True)
    s = tot
    sh = 1
    while sh < _BC:
        s = s + jnp.concatenate(
            [jnp.zeros((1, sh), jnp.int32), s[:, :-sh]], axis=1)
        sh *= 2
    ex_n = s - tot + carry_ref[0, 0]
    run = jnp.zeros((1, _BC), jnp.int32)
    rows = []
    for t in range(_NTILES):
        rows.append(run)
        run = run + h[t:t + 1]
    ex_t = jnp.concatenate(rows, axis=0)
    bases_ref[...] = ex_t + ex_n
    csr_ref[...] = ex_n
    cnt_ref[...] = tot
    carry_ref[0, 0] = carry_ref[0, 0] + jnp.sum(h)


def _tc_offsets(hist):
    return pl.pallas_call(
        _offsets_body,
        grid=(_NPAD // _BC,),
        in_specs=[pl.BlockSpec((_NTILES, _BC), lambda j: (0, j))],
        out_specs=[pl.BlockSpec((_NTILES, _BC), lambda j: (0, j)),
                   pl.BlockSpec((1, _BC), lambda j: (0, j)),
                   pl.BlockSpec((1, _BC), lambda j: (0, j))],
        out_shape=(jax.ShapeDtypeStruct((_NTILES, _NPAD), jnp.int32),
                   jax.ShapeDtypeStruct((1, _NPAD), jnp.int32),
                   jax.ShapeDtypeStruct((1, _NPAD), jnp.int32)),
        scratch_shapes=[pltpu.SMEM((1, 1), jnp.int32)],
    )(hist)


def _sc_place_body(col_hbm, attr_hbm, bases_hbm, rows_hbm,
                   next_v, col_v, attr_v, slots_v, tmp_v):
    wid = lax.axis_index("s") * 2 + lax.axis_index("c")
    pltpu.sync_copy(bases_hbm.at[wid], next_v)
    base_e = wid * _EPT
    @pl.loop(0, _EPT, step=_CH3)
    def _(e0):
        pltpu.sync_copy(col_hbm.at[pl.ds(base_e + e0, _CH3)], col_v)
        pltpu.sync_copy(attr_hbm.at[pl.ds(base_e + e0, _CH3)], attr_v)
        @pl.loop(0, _CH3, step=16)
        def _(v):
            cv = col_v[pl.ds(v, 16)]
            sk, sv, rank, last = _vec_ranks(cv)
            base = plsc.load_gather(next_v, [sk])
            slot = base + rank
            plsc.store_scatter(next_v, [sk], slot + 1, mask=last)
            plsc.store_scatter(tmp_v, [sv], slot)
            slots_v[pl.ds(v, 16)] = tmp_v[...]
        pltpu.sync_copy(attr_v, rows_hbm.at[slots_v])


def _sc_place(col, attr, bases):
    mesh = plsc.VectorSubcoreMesh(core_axis_name="c", subcore_axis_name="s")
    run = pl.kernel(
        _sc_place_body,
        mesh=mesh,
        compiler_params=_SC_PARAMS_NL,
        out_type=jax.ShapeDtypeStruct((_EPAD, DE), jnp.float32),
        scratch_types=[
            pltpu.VMEM((_NPAD,), jnp.int32),
            pltpu.VMEM((_CH3,), jnp.int32),
            pltpu.VMEM((_CH3, DE), jnp.float32),
            pltpu.VMEM((_CH3,), jnp.int32),
            pltpu.VMEM((16,), jnp.int32),
        ],
    )
    return run(col, attr, bases)


_NG = 196


def _sc_segmax_body(rows_hbm, csr_hbm, cnt_hbm, omax_hbm,
                    rows_v, stage_v, cntc_v, seg_v):
    wid = lax.axis_index("s") * 2 + lax.axis_index("c")
    n0 = wid * _NPT32
    lane = lax.iota(jnp.int32, 16)
    cntc_v[pl.ds(_NPT32 - 8, 16)] = jnp.zeros((16,), jnp.int32)
    pltpu.sync_copy(cnt_hbm.at[0].at[pl.ds(n0, _NPT32)],
                    cntc_v.at[pl.ds(0, _NPT32)])
    pltpu.sync_copy(csr_hbm.at[0].at[pl.ds(n0, 16)], seg_v)
    seg0 = jnp.sum(jnp.where(lane == 0, seg_v[...], 0))
    base_al = seg0 // 8 * 8
    pltpu.sync_copy(rows_hbm.at[pl.ds(base_al, _WIN)], rows_v)
    p0 = seg0 - base_al
    neg = jnp.full((16,), -jnp.inf, jnp.float32)
    zro = jnp.full((16,), 0.0, jnp.float32)

    def group_body(g, carry):
        p, wcur = carry
        cvec = cntc_v[pl.ds(g * 16, 16)]

        def node_body(i, nc):
            p, wcur = nc
            k = jnp.sum(jnp.where(lane == i, cvec, 0))
            acc0 = jnp.where(k > 0, neg, zro)

            def edge_body(_j, ec):
                pe, we, acc = ec
                refill = pe == _WIN
                wnew = jnp.where(refill, we + _WIN, we)
                @pl.when(refill)
                def _():
                    pltpu.sync_copy(rows_hbm.at[pl.ds(wnew, _WIN)], rows_v)
                p2 = jnp.where(refill, 0, pe)
                acc = jnp.maximum(acc, rows_v[p2])
                return (p2 + 1, wnew, acc)

            p, wcur, acc = lax.fori_loop(0, k, edge_body, (p, wcur, acc0))
            stage_v[g * 16 + i] = acc
            return (p, wcur)

        return lax.fori_loop(0, 16, node_body, (p, wcur))

    lax.fori_loop(0, _NG, group_body, (p0, base_al))
    pltpu.sync_copy(stage_v.at[pl.ds(0, _NPT32)],
                    omax_hbm.at[pl.ds(n0, _NPT32)])


def _sc_segmax(rows, csr, cnt):
    mesh = plsc.VectorSubcoreMesh(core_axis_name="c", subcore_axis_name="s")
    run = pl.kernel(
        _sc_segmax_body,
        mesh=mesh,
        compiler_params=_SC_PARAMS_NL,
        out_type=jax.ShapeDtypeStruct((_NPAD, DE), jnp.float32),
        scratch_types=[
            pltpu.VMEM((_WIN, DE), jnp.float32),
            pltpu.VMEM((_NG * 16, DE), jnp.float32),
            pltpu.VMEM((_NG * 16,), jnp.int32),
            pltpu.VMEM((16,), jnp.int32),
        ],
    )
    return run(rows, csr, cnt)


_BLK = 2000


def _mlp_body(x_ref, s_ref, c_ref, m_ref, b_ref, u_ref, w1_ref, b1_ref,
              w2_ref, b2_ref, o_ref):
    x = x_ref[...]
    s = s_ref[0] + s_ref[1]
    c = c_ref[0] + c_ref[1]
    m = jnp.where(c > 0, m_ref[...], 0.0)
    mn = s / jnp.maximum(c, 1.0)
    oh = (b_ref[...] == jax.lax.broadcasted_iota(jnp.int32, (_BLK, G), 1))
    ub = jnp.dot(oh.astype(jnp.float32), u_ref[...],
                 preferred_element_type=jnp.float32)
    cat = jnp.concatenate([x, s, m, mn, ub], axis=1)
    h = jnp.dot(cat, w1_ref[...], preferred_element_type=jnp.float32)
    h = jnp.maximum(h + b1_ref[...], 0.0)
    o = jnp.dot(h, w2_ref[...], preferred_element_type=jnp.float32)
    o_ref[...] = o + b2_ref[...] + x


def _mlp(x, sum2, cnt2, maxraw, batch2d, u, W1, b1, W2, b2):
    nblk = N // _BLK
    rep = lambda i: (0, 0)
    return pl.pallas_call(
        _mlp_body,
        grid=(nblk,),
        in_specs=[
            pl.BlockSpec((_BLK, D), lambda i: (i, 0)),
            pl.BlockSpec((2, _BLK, DE), lambda i: (0, i, 0)),
            pl.BlockSpec((2, _BLK, 1), lambda i: (0, i, 0)),
            pl.BlockSpec((_BLK, DE), lambda i: (i, 0)),
            pl.BlockSpec((_BLK, 1), lambda i: (i, 0)),
            pl.BlockSpec((G, 16), rep),
            pl.BlockSpec((IN_DIM, H), rep),
            pl.BlockSpec((1, H), rep),
            pl.BlockSpec((H, D), rep),
            pl.BlockSpec((1, D), rep),
        ],
        out_specs=pl.BlockSpec((_BLK, D), lambda i: (i, 0)),
        out_shape=jax.ShapeDtypeStruct((N, D), jnp.float32),
    )(x, sum2, cnt2, maxraw, batch2d, u, W1, b1, W2, b2)


def kernel(x, edge_index, edge_attr, u, batch, W1, b1, W2, b2):
    col = edge_index[1]
    sum2, cnt2 = _sc_sumcnt(col, edge_attr)
    cnt2 = cnt2.reshape(2, _NPAD, 1)
    hist = _sc_hist(col)
    bases, csr, cnt = _tc_offsets(hist)
    rows = _sc_place(col, edge_attr, bases)
    maxraw = _sc_segmax(rows, csr, cnt)
    return _mlp(x, sum2, cnt2, maxraw,
                batch.reshape(N, 1), u, W1, b1.reshape(1, H), W2,
                b2.reshape(1, D))


# speedup vs baseline: 7.2053x; 1.1164x over previous
import dataclasses
import functools

import jax
import jax.numpy as jnp
from jax import lax
from jax.experimental import pallas as pl
from jax.experimental.pallas import tpu as pltpu
from jax.experimental.pallas import tpu_sc as plsc

N = 100000
E = 3200000
D = 128
DE = 16
G = 8
H = 256
IN_DIM = D + 3 * DE + 16

_NTILES = 32
_EPT = E // _NTILES
_CH = 1000
_NPAD = 100096
_NPT = _NPAD // 16
_ZCH = 368
_CH3 = 800
_NPT32 = _NPAD // 32
_WIN = 1000
_EPAD = E + _WIN + 8

_SC_PARAMS = pltpu.CompilerParams(use_tc_tiling_on_sc=False)
_SC_PARAMS_NL = (
    dataclasses.replace(_SC_PARAMS, needs_layout_passes=False)
    if "needs_layout_passes" in pltpu.CompilerParams.__dataclass_fields__
    else _SC_PARAMS)


def _lane_take(x, idx):
    dnums = lax.GatherDimensionNumbers(
        offset_dims=(), collapsed_slice_dims=(0,), start_index_map=(0,))
    return lax.gather(x, idx[:, None], dnums, (1,),
                      mode=lax.GatherScatterMode.PROMISE_IN_BOUNDS)


def _vec_ranks(cv):
    lane = lax.iota(jnp.int32, 16)
    sk, sv = plsc.sort_key_val(cv, lane)
    prev = _lane_take(sk, jnp.maximum(lane - 1, 0))
    bnd = jnp.logical_or(lane == 0, sk != prev)
    start = plsc.cummax(jnp.where(bnd, lane, 0))
    rank = lane - start
    nxt = _lane_take(sk, jnp.minimum(lane + 1, 15))
    last = jnp.logical_or(lane == 15, sk != nxt)
    return sk, sv, rank, last


def _sc_hist_body(col_hbm, zi_hbm, hist_hbm, hist_v, col_v):
    wid = lax.axis_index("s") * 2 + lax.axis_index("c")
    @pl.loop(0, _NPAD, step=_NPT)
    def _(r):
        pltpu.sync_copy(zi_hbm, hist_v.at[pl.ds(r, _NPT)])
    base_e = wid * _EPT
    @pl.loop(0, _EPT, step=_CH3)
    def _(e0):
        pltpu.sync_copy(col_hbm.at[pl.ds(base_e + e0, _CH3)], col_v)
        @pl.loop(0, _CH3, step=16)
        def _(v):
            cv = col_v[pl.ds(v, 16)]
            sk, _sv, rank, last = _vec_ranks(cv)
            plsc.addupdate_scatter(hist_v, [sk], rank + 1, mask=last)
    pltpu.sync_copy(hist_v, hist_hbm.at[wid])


def _sc_hist(col):
    mesh = plsc.VectorSubcoreMesh(core_axis_name="c", subcore_axis_name="s")
    run = pl.kernel(
        _sc_hist_body,
        mesh=mesh,
        compiler_params=_SC_PARAMS_NL,
        out_type=jax.ShapeDtypeStruct((_NTILES, _NPAD), jnp.int32),
        scratch_types=[
            pltpu.VMEM((_NPAD,), jnp.int32),
            pltpu.VMEM((_CH3,), jnp.int32),
        ],
    )
    return run(col, jnp.zeros((_NPT,), jnp.int32))


_BC = 4352


def _offsets_body(h_ref, bases_ref, csr_ref, cnt_ref, carry_ref):
    @pl.when(pl.program_id(0) == 0)
    def _():
        carry_ref[0, 0] = 0
    h = h_ref[...]
    tot = jnp.sum(h, axis=0, keepdims=True)
    s = tot
    sh = 1
    while sh < _BC:
        s = s + jnp.concatenate(
            [jnp.zeros((1, sh), jnp.int32), s[:, :-sh]], axis=1)
        sh *= 2
    ex_n = s - tot + carry_ref[0, 0]
    run = jnp.zeros((1, _BC), jnp.int32)
    rows = []
    for t in range(_NTILES):
        rows.append(run)
        run = run + h[t:t + 1]
    ex_t = jnp.concatenate(rows, axis=0)
    bases_ref[...] = ex_t + ex_n
    csr_ref[...] = ex_n
    cnt_ref[...] = tot
    carry_ref[0, 0] = carry_ref[0, 0] + jnp.sum(h)


def _tc_offsets(hist):
    return pl.pallas_call(
        _offsets_body,
        grid=(_NPAD // _BC,),
        in_specs=[pl.BlockSpec((_NTILES, _BC), lambda j: (0, j))],
        out_specs=[pl.BlockSpec((_NTILES, _BC), lambda j: (0, j)),
                   pl.BlockSpec((1, _BC), lambda j: (0, j)),
                   pl.BlockSpec((1, _BC), lambda j: (0, j))],
        out_shape=(jax.ShapeDtypeStruct((_NTILES, _NPAD), jnp.int32),
                   jax.ShapeDtypeStruct((1, _NPAD), jnp.int32),
                   jax.ShapeDtypeStruct((1, _NPAD), jnp.int32)),
        scratch_shapes=[pltpu.SMEM((1, 1), jnp.int32)],
    )(hist)


def _sc_place_body(col_hbm, attr_hbm, bases_hbm, rows_hbm,
                   next_v, col_v, attr_v, slots_v, tmp_v):
    wid = lax.axis_index("s") * 2 + lax.axis_index("c")
    pltpu.sync_copy(bases_hbm.at[wid], next_v)
    base_e = wid * _EPT
    @pl.loop(0, _EPT, step=_CH3)
    def _(e0):
        pltpu.sync_copy(col_hbm.at[pl.ds(base_e + e0, _CH3)], col_v)
        pltpu.sync_copy(attr_hbm.at[pl.ds(base_e + e0, _CH3)], attr_v)
        @pl.loop(0, _CH3, step=16)
        def _(v):
            cv = col_v[pl.ds(v, 16)]
            sk, sv, rank, last = _vec_ranks(cv)
            base = plsc.load_gather(next_v, [sk])
            slot = base + rank
            plsc.store_scatter(next_v, [sk], slot + 1, mask=last)
            plsc.store_scatter(tmp_v, [sv], slot)
            slots_v[pl.ds(v, 16)] = tmp_v[...]
        pltpu.sync_copy(attr_v, rows_hbm.at[slots_v])


def _sc_place(col, attr, bases):
    mesh = plsc.VectorSubcoreMesh(core_axis_name="c", subcore_axis_name="s")
    run = pl.kernel(
        _sc_place_body,
        mesh=mesh,
        compiler_params=_SC_PARAMS_NL,
        out_type=jax.ShapeDtypeStruct((_EPAD, DE), jnp.float32),
        scratch_types=[
            pltpu.VMEM((_NPAD,), jnp.int32),
            pltpu.VMEM((_CH3,), jnp.int32),
            pltpu.VMEM((_CH3, DE), jnp.float32),
            pltpu.VMEM((_CH3,), jnp.int32),
            pltpu.VMEM((16,), jnp.int32),
        ],
    )
    return run(col, attr, bases)


_NG = 196


def _sc_reduce_body(rows_hbm, csr_hbm, cnt_hbm, omax_hbm, osum_hbm, ocnf_hbm,
                    rows_v, stage_m, stage_s, cntc_v, cntf_v, seg_v):
    wid = lax.axis_index("s") * 2 + lax.axis_index("c")
    n0 = wid * _NPT32
    lane = lax.iota(jnp.int32, 16)
    cntc_v[pl.ds(_NPT32 - 8, 16)] = jnp.zeros((16,), jnp.int32)
    pltpu.sync_copy(cnt_hbm.at[0].at[pl.ds(n0, _NPT32)],
                    cntc_v.at[pl.ds(0, _NPT32)])
    pltpu.sync_copy(csr_hbm.at[0].at[pl.ds(n0, 16)], seg_v)
    seg0 = jnp.sum(jnp.where(lane == 0, seg_v[...], 0))
    base_al = seg0 // 8 * 8
    pltpu.sync_copy(rows_hbm.at[pl.ds(base_al, _WIN)], rows_v)
    p0 = seg0 - base_al
    neg = jnp.full((16,), -jnp.inf, jnp.float32)
    zro = jnp.full((16,), 0.0, jnp.float32)

    def group_body(g, carry):
        p, wcur = carry
        cvec = cntc_v[pl.ds(g * 16, 16)]
        cntf_v[pl.ds(g * 16, 16)] = cvec.astype(jnp.float32)

        def node_body(i, nc):
            p, wcur = nc
            k = jnp.sum(jnp.where(lane == i, cvec, 0))
            acc0m = jnp.where(k > 0, neg, zro)

            def edge_body(_j, ec):
                pe, we, accm, accs = ec
                refill = pe == _WIN
                wnew = jnp.where(refill, we + _WIN, we)
                @pl.when(refill)
                def _():
                    pltpu.sync_copy(rows_hbm.at[pl.ds(wnew, _WIN)], rows_v)
                p2 = jnp.where(refill, 0, pe)
                row = rows_v[p2]
                return (p2 + 1, wnew, jnp.maximum(accm, row), accs + row)

            p, wcur, accm, accs = lax.fori_loop(
                0, k, edge_body, (p, wcur, acc0m, zro))
            stage_m[g * 16 + i] = accm
            stage_s[g * 16 + i] = accs
            return (p, wcur)

        return lax.fori_loop(0, 16, node_body, (p, wcur))

    lax.fori_loop(0, _NG, group_body, (p0, base_al))
    pltpu.sync_copy(stage_m.at[pl.ds(0, _NPT32)],
                    omax_hbm.at[pl.ds(n0, _NPT32)])
    pltpu.sync_copy(stage_s.at[pl.ds(0, _NPT32)],
                    osum_hbm.at[pl.ds(n0, _NPT32)])
    pltpu.sync_copy(cntf_v.at[pl.ds(0, _NPT32)],
                    ocnf_hbm.at[pl.ds(n0, _NPT32)])


def _sc_reduce(rows, csr, cnt):
    mesh = plsc.VectorSubcoreMesh(core_axis_name="c", subcore_axis_name="s")
    run = pl.kernel(
        _sc_reduce_body,
        mesh=mesh,
        compiler_params=_SC_PARAMS_NL,
        out_type=(jax.ShapeDtypeStruct((_NPAD, DE), jnp.float32),
                  jax.ShapeDtypeStruct((_NPAD, DE), jnp.float32),
                  jax.ShapeDtypeStruct((_NPAD,), jnp.float32)),
        scratch_types=[
            pltpu.VMEM((_WIN, DE), jnp.float32),
            pltpu.VMEM((_NG * 16, DE), jnp.float32),
            pltpu.VMEM((_NG * 16, DE), jnp.float32),
            pltpu.VMEM((_NG * 16,), jnp.int32),
            pltpu.VMEM((_NG * 16,), jnp.float32),
            pltpu.VMEM((16,), jnp.int32),
        ],
    )
    return run(rows, csr, cnt)


_BLK = 2000


def _mlp_body(x_ref, s_ref, c_ref, m_ref, b_ref, u_ref, w1_ref, b1_ref,
              w2_ref, b2_ref, o_ref):
    x = x_ref[...]
    s = s_ref[...]
    c = c_ref[...]
    m = jnp.where(c > 0, m_ref[...], 0.0)
    mn = s / jnp.maximum(c, 1.0)
    oh = (b_ref[...] == jax.lax.broadcasted_iota(jnp.int32, (_BLK, G), 1))
    ub = jnp.dot(oh.astype(jnp.float32), u_ref[...],
                 preferred_element_type=jnp.float32)
    cat = jnp.concatenate([x, s, m, mn, ub], axis=1)
    h = jnp.dot(cat, w1_ref[...], preferred_element_type=jnp.float32)
    h = jnp.maximum(h + b1_ref[...], 0.0)
    o = jnp.dot(h, w2_ref[...], preferred_element_type=jnp.float32)
    o_ref[...] = o + b2_ref[...] + x


def _mlp(x, sum2, cnt2, maxraw, batch2d, u, W1, b1, W2, b2):
    nblk = N // _BLK
    rep = lambda i: (0, 0)
    return pl.pallas_call(
        _mlp_body,
        grid=(nblk,),
        in_specs=[
            pl.BlockSpec((_BLK, D), lambda i: (i, 0)),
            pl.BlockSpec((_BLK, DE), lambda i: (i, 0)),
            pl.BlockSpec((_BLK, 1), lambda i: (i, 0)),
            pl.BlockSpec((_BLK, DE), lambda i: (i, 0)),
            pl.BlockSpec((_BLK, 1), lambda i: (i, 0)),
            pl.BlockSpec((G, 16), rep),
            pl.BlockSpec((IN_DIM, H), rep),
            pl.BlockSpec((1, H), rep),
            pl.BlockSpec((H, D), rep),
            pl.BlockSpec((1, D), rep),
        ],
        out_specs=pl.BlockSpec((_BLK, D), lambda i: (i, 0)),
        out_shape=jax.ShapeDtypeStruct((N, D), jnp.float32),
    )(x, sum2, cnt2, maxraw, batch2d, u, W1, b1, W2, b2)


def kernel(x, edge_index, edge_attr, u, batch, W1, b1, W2, b2):
    col = edge_index[1]
    hist = _sc_hist(col)
    bases, csr, cnt = _tc_offsets(hist)
    rows = _sc_place(col, edge_attr, bases)
    maxraw, sumv, cnf = _sc_reduce(rows, csr, cnt)
    return _mlp(x, sumv, cnf.reshape(_NPAD, 1), maxraw,
                batch.reshape(N, 1), u, W1, b1.reshape(1, H), W2,
                b2.reshape(1, D))


# speedup vs baseline: 7.6911x; 1.0674x over previous
import dataclasses
import functools

import jax
import jax.numpy as jnp
from jax import lax
from jax.experimental import pallas as pl
from jax.experimental.pallas import tpu as pltpu
from jax.experimental.pallas import tpu_sc as plsc

N = 100000
E = 3200000
D = 128
DE = 16
G = 8
H = 256
IN_DIM = D + 3 * DE + 16

_NTILES = 32
_EPT = E // _NTILES
_CH = 1000
_NPAD = 100096
_NPT = _NPAD // 16
_ZCH = 368
_CH3 = 800
_NPT32 = _NPAD // 32
_WIN = 1000
_EPAD = E + _WIN + 8

_SC_PARAMS = pltpu.CompilerParams(use_tc_tiling_on_sc=False)
_SC_PARAMS_NL = (
    dataclasses.replace(_SC_PARAMS, needs_layout_passes=False)
    if "needs_layout_passes" in pltpu.CompilerParams.__dataclass_fields__
    else _SC_PARAMS)


def _lane_take(x, idx):
    dnums = lax.GatherDimensionNumbers(
        offset_dims=(), collapsed_slice_dims=(0,), start_index_map=(0,))
    return lax.gather(x, idx[:, None], dnums, (1,),
                      mode=lax.GatherScatterMode.PROMISE_IN_BOUNDS)


def _vec_ranks(cv):
    lane = lax.iota(jnp.int32, 16)
    sk, sv = plsc.sort_key_val(cv, lane)
    prev = _lane_take(sk, jnp.maximum(lane - 1, 0))
    bnd = jnp.logical_or(lane == 0, sk != prev)
    start = plsc.cummax(jnp.where(bnd, lane, 0))
    rank = lane - start
    nxt = _lane_take(sk, jnp.minimum(lane + 1, 15))
    last = jnp.logical_or(lane == 15, sk != nxt)
    return sk, sv, rank, last


def _sc_hist_body(col_hbm, zi_hbm, hist_hbm, hist_v, col_v):
    wid = lax.axis_index("s") * 2 + lax.axis_index("c")
    @pl.loop(0, _NPAD, step=_NPT)
    def _(r):
        pltpu.sync_copy(zi_hbm, hist_v.at[pl.ds(r, _NPT)])
    base_e = wid * _EPT
    @pl.loop(0, _EPT, step=_CH3)
    def _(e0):
        pltpu.sync_copy(col_hbm.at[pl.ds(base_e + e0, _CH3)], col_v)
        @pl.loop(0, _CH3, step=16)
        def _(v):
            cv = col_v[pl.ds(v, 16)]
            sk, _sv, rank, last = _vec_ranks(cv)
            plsc.addupdate_scatter(hist_v, [sk], rank + 1, mask=last)
    pltpu.sync_copy(hist_v, hist_hbm.at[wid])


def _sc_hist(col):
    mesh = plsc.VectorSubcoreMesh(core_axis_name="c", subcore_axis_name="s")
    run = pl.kernel(
        _sc_hist_body,
        mesh=mesh,
        compiler_params=_SC_PARAMS_NL,
        out_type=jax.ShapeDtypeStruct((_NTILES, _NPAD), jnp.int32),
        scratch_types=[
            pltpu.VMEM((_NPAD,), jnp.int32),
            pltpu.VMEM((_CH3,), jnp.int32),
        ],
    )
    return run(col, jnp.zeros((_NPT,), jnp.int32))


_BC = 4352


def _offsets_body(h_ref, bases_ref, csr_ref, cnt_ref, carry_ref):
    @pl.when(pl.program_id(0) == 0)
    def _():
        carry_ref[0, 0] = 0
    h = h_ref[...]
    tot = jnp.sum(h, axis=0, keepdims=True)
    s = tot
    sh = 1
    while sh < _BC:
        s = s + jnp.concatenate(
            [jnp.zeros((1, sh), jnp.int32), s[:, :-sh]], axis=1)
        sh *= 2
    ex_n = s - tot + carry_ref[0, 0]
    run = jnp.zeros((1, _BC), jnp.int32)
    rows = []
    for t in range(_NTILES):
        rows.append(run)
        run = run + h[t:t + 1]
    ex_t = jnp.concatenate(rows, axis=0)
    bases_ref[...] = ex_t + ex_n
    csr_ref[...] = ex_n
    cnt_ref[...] = tot
    carry_ref[0, 0] = carry_ref[0, 0] + jnp.sum(h)


def _tc_offsets(hist):
    return pl.pallas_call(
        _offsets_body,
        grid=(_NPAD // _BC,),
        in_specs=[pl.BlockSpec((_NTILES, _BC), lambda j: (0, j))],
        out_specs=[pl.BlockSpec((_NTILES, _BC), lambda j: (0, j)),
                   pl.BlockSpec((1, _BC), lambda j: (0, j)),
                   pl.BlockSpec((1, _BC), lambda j: (0, j))],
        out_shape=(jax.ShapeDtypeStruct((_NTILES, _NPAD), jnp.int32),
                   jax.ShapeDtypeStruct((1, _NPAD), jnp.int32),
                   jax.ShapeDtypeStruct((1, _NPAD), jnp.int32)),
        scratch_shapes=[pltpu.SMEM((1, 1), jnp.int32)],
    )(hist)


def _sc_place_body(col_hbm, attr_hbm, bases_hbm, rows_hbm,
                   next_v, col_v, attr_v, slots_v, tmp_v):
    wid = lax.axis_index("s") * 2 + lax.axis_index("c")
    pltpu.sync_copy(bases_hbm.at[wid], next_v)
    base_e = wid * _EPT
    @pl.loop(0, _EPT, step=_CH3)
    def _(e0):
        pltpu.sync_copy(col_hbm.at[pl.ds(base_e + e0, _CH3)], col_v)
        pltpu.sync_copy(attr_hbm.at[pl.ds(base_e + e0, _CH3)], attr_v)
        @pl.loop(0, _CH3, step=16)
        def _(v):
            cv = col_v[pl.ds(v, 16)]
            sk, sv, rank, last = _vec_ranks(cv)
            base = plsc.load_gather(next_v, [sk])
            slot = base + rank
            plsc.store_scatter(next_v, [sk], slot + 1, mask=last)
            plsc.store_scatter(tmp_v, [sv], slot)
            slots_v[pl.ds(v, 16)] = tmp_v[...]
        pltpu.sync_copy(attr_v, rows_hbm.at[slots_v])


def _sc_place(col, attr, bases):
    mesh = plsc.VectorSubcoreMesh(core_axis_name="c", subcore_axis_name="s")
    run = pl.kernel(
        _sc_place_body,
        mesh=mesh,
        compiler_params=_SC_PARAMS_NL,
        out_type=jax.ShapeDtypeStruct((_EPAD, DE), jnp.float32),
        scratch_types=[
            pltpu.VMEM((_NPAD,), jnp.int32),
            pltpu.VMEM((_CH3,), jnp.int32),
            pltpu.VMEM((_CH3, DE), jnp.float32),
            pltpu.VMEM((_CH3,), jnp.int32),
            pltpu.VMEM((16,), jnp.int32),
        ],
    )
    return run(col, attr, bases)


_NG = 196


def _sc_reduce_body(rows_hbm, csr_hbm, cnt_hbm, omax_hbm, osum_hbm, ocnf_hbm,
                    rows_v, stage_m, stage_s, cntc_v, cntf_v, seg_v):
    wid = lax.axis_index("s") * 2 + lax.axis_index("c")
    n0 = wid * _NPT32
    lane = lax.iota(jnp.int32, 16)
    cntc_v[pl.ds(_NPT32 - 8, 16)] = jnp.zeros((16,), jnp.int32)
    pltpu.sync_copy(cnt_hbm.at[0].at[pl.ds(n0, _NPT32)],
                    cntc_v.at[pl.ds(0, _NPT32)])
    pltpu.sync_copy(csr_hbm.at[0].at[pl.ds(n0, 16)], seg_v)
    seg0 = jnp.sum(jnp.where(lane == 0, seg_v[...], 0))
    base_al = seg0 // 8 * 8
    pltpu.sync_copy(rows_hbm.at[pl.ds(base_al, _WIN)], rows_v)
    p0 = seg0 - base_al
    neg = jnp.full((16,), -jnp.inf, jnp.float32)
    zro = jnp.full((16,), 0.0, jnp.float32)

    def group_body(g, carry):
        p, wcur = carry
        cvec = cntc_v[pl.ds(g * 16, 16)]
        cntf_v[pl.ds(g * 16, 16)] = cvec.astype(jnp.float32)

        def node_body(i, nc):
            p, wcur = nc
            k = jnp.sum(jnp.where(lane == i, cvec, 0))
            acc0m = jnp.where(k > 0, neg, zro)

            def run_cond(st):
                krem, _p, _w, _am, _as_ = st
                return krem > 0

            def run_body(st):
                krem, p, wcur, accm, accs = st
                refill = p == _WIN
                wcur = jnp.where(refill, wcur + _WIN, wcur)
                @pl.when(refill)
                def _():
                    pltpu.sync_copy(rows_hbm.at[pl.ds(wcur, _WIN)], rows_v)
                p = jnp.where(refill, 0, p)
                r = jnp.minimum(krem, _WIN - p)

                def pair_body(_j, ec):
                    pe, am, asum = ec
                    r0 = rows_v[pe]
                    r1 = rows_v[pe + 1]
                    return (pe + 2, jnp.maximum(am, jnp.maximum(r0, r1)),
                            asum + r0 + r1)

                def one_body(_j, ec):
                    pe, am, asum = ec
                    r0 = rows_v[pe]
                    return (pe + 1, jnp.maximum(am, r0), asum + r0)

                p, accm, accs = lax.fori_loop(0, r // 2, pair_body,
                                              (p, accm, accs))
                p, accm, accs = lax.fori_loop(0, r % 2, one_body,
                                              (p, accm, accs))
                return (krem - r, p, wcur, accm, accs)

            _k, p, wcur, accm, accs = lax.while_loop(
                run_cond, run_body, (k, p, wcur, acc0m, zro))
            stage_m[g * 16 + i] = accm
            stage_s[g * 16 + i] = accs
            return (p, wcur)

        return lax.fori_loop(0, 16, node_body, (p, wcur))

    lax.fori_loop(0, _NG, group_body, (p0, base_al))
    pltpu.sync_copy(stage_m.at[pl.ds(0, _NPT32)],
                    omax_hbm.at[pl.ds(n0, _NPT32)])
    pltpu.sync_copy(stage_s.at[pl.ds(0, _NPT32)],
                    osum_hbm.at[pl.ds(n0, _NPT32)])
    pltpu.sync_copy(cntf_v.at[pl.ds(0, _NPT32)],
                    ocnf_hbm.at[pl.ds(n0, _NPT32)])


def _sc_reduce(rows, csr, cnt):
    mesh = plsc.VectorSubcoreMesh(core_axis_name="c", subcore_axis_name="s")
    run = pl.kernel(
        _sc_reduce_body,
        mesh=mesh,
        compiler_params=_SC_PARAMS_NL,
        out_type=(jax.ShapeDtypeStruct((_NPAD, DE), jnp.float32),
                  jax.ShapeDtypeStruct((_NPAD, DE), jnp.float32),
                  jax.ShapeDtypeStruct((_NPAD,), jnp.float32)),
        scratch_types=[
            pltpu.VMEM((_WIN, DE), jnp.float32),
            pltpu.VMEM((_NG * 16, DE), jnp.float32),
            pltpu.VMEM((_NG * 16, DE), jnp.float32),
            pltpu.VMEM((_NG * 16,), jnp.int32),
            pltpu.VMEM((_NG * 16,), jnp.float32),
            pltpu.VMEM((16,), jnp.int32),
        ],
    )
    return run(rows, csr, cnt)


_BLK = 2000


def _mlp_body(x_ref, s_ref, c_ref, m_ref, b_ref, u_ref, w1_ref, b1_ref,
              w2_ref, b2_ref, o_ref):
    x = x_ref[...]
    s = s_ref[...]
    c = c_ref[...]
    m = jnp.where(c > 0, m_ref[...], 0.0)
    mn = s / jnp.maximum(c, 1.0)
    oh = (b_ref[...] == jax.lax.broadcasted_iota(jnp.int32, (_BLK, G), 1))
    ub = jnp.dot(oh.astype(jnp.float32), u_ref[...],
                 preferred_element_type=jnp.float32)
    cat = jnp.concatenate([x, s, m, mn, ub], axis=1)
    h = jnp.dot(cat, w1_ref[...], preferred_element_type=jnp.float32)
    h = jnp.maximum(h + b1_ref[...], 0.0)
    o = jnp.dot(h, w2_ref[...], preferred_element_type=jnp.float32)
    o_ref[...] = o + b2_ref[...] + x


def _mlp(x, sum2, cnt2, maxraw, batch2d, u, W1, b1, W2, b2):
    nblk = N // _BLK
    rep = lambda i: (0, 0)
    return pl.pallas_call(
        _mlp_body,
        grid=(nblk,),
        in_specs=[
            pl.BlockSpec((_BLK, D), lambda i: (i, 0)),
            pl.BlockSpec((_BLK, DE), lambda i: (i, 0)),
            pl.BlockSpec((_BLK, 1), lambda i: (i, 0)),
            pl.BlockSpec((_BLK, DE), lambda i: (i, 0)),
            pl.BlockSpec((_BLK, 1), lambda i: (i, 0)),
            pl.BlockSpec((G, 16), rep),
            pl.BlockSpec((IN_DIM, H), rep),
            pl.BlockSpec((1, H), rep),
            pl.BlockSpec((H, D), rep),
            pl.BlockSpec((1, D), rep),
        ],
        out_specs=pl.BlockSpec((_BLK, D), lambda i: (i, 0)),
        out_shape=jax.ShapeDtypeStruct((N, D), jnp.float32),
    )(x, sum2, cnt2, maxraw, batch2d, u, W1, b1, W2, b2)


def kernel(x, edge_index, edge_attr, u, batch, W1, b1, W2, b2):
    col = edge_index[1]
    hist = _sc_hist(col)
    bases, csr, cnt = _tc_offsets(hist)
    rows = _sc_place(col, edge_attr, bases)
    maxraw, sumv, cnf = _sc_reduce(rows, csr, cnt)
    return _mlp(x, sumv, cnf.reshape(_NPAD, 1), maxraw,
                batch.reshape(N, 1), u, W1, b1.reshape(1, H), W2,
                b2.reshape(1, D))


# speedup vs baseline: 8.4911x; 1.1040x over previous
import dataclasses
import functools

import jax
import jax.numpy as jnp
from jax import lax
from jax.experimental import pallas as pl
from jax.experimental.pallas import tpu as pltpu
from jax.experimental.pallas import tpu_sc as plsc

N = 100000
E = 3200000
D = 128
DE = 16
G = 8
H = 256
IN_DIM = D + 3 * DE + 16

_NTILES = 32
_EPT = E // _NTILES
_CH = 1000
_NPAD = 100096
_NPT = _NPAD // 16
_ZCH = 368
_CH3 = 800
_NPT32 = _NPAD // 32
_WIN = 1000
_EPAD = E + _WIN + 8

_SC_PARAMS = pltpu.CompilerParams(use_tc_tiling_on_sc=False)
_SC_PARAMS_NL = (
    dataclasses.replace(_SC_PARAMS, needs_layout_passes=False)
    if "needs_layout_passes" in pltpu.CompilerParams.__dataclass_fields__
    else _SC_PARAMS)


def _lane_take(x, idx):
    dnums = lax.GatherDimensionNumbers(
        offset_dims=(), collapsed_slice_dims=(0,), start_index_map=(0,))
    return lax.gather(x, idx[:, None], dnums, (1,),
                      mode=lax.GatherScatterMode.PROMISE_IN_BOUNDS)


def _vec_ranks(cv):
    lane = lax.iota(jnp.int32, 16)
    sk, sv = plsc.sort_key_val(cv, lane)
    prev = _lane_take(sk, jnp.maximum(lane - 1, 0))
    bnd = jnp.logical_or(lane == 0, sk != prev)
    start = plsc.cummax(jnp.where(bnd, lane, 0))
    rank = lane - start
    nxt = _lane_take(sk, jnp.minimum(lane + 1, 15))
    last = jnp.logical_or(lane == 15, sk != nxt)
    return sk, sv, rank, last


def _hist_chunk(col_v, hist_v):
    @pl.loop(0, _CH3, step=16)
    def _(v):
        cv = col_v[pl.ds(v, 16)]
        sk, _sv, rank, last = _vec_ranks(cv)
        plsc.addupdate_scatter(hist_v, [sk], rank + 1, mask=last)


def _sc_hist_body(col_hbm, zi_hbm, hist_hbm, hist_v, col_v0, col_v1,
                  sem0, sem1):
    wid = lax.axis_index("s") * 2 + lax.axis_index("c")
    @pl.loop(0, _NPAD, step=_NPT)
    def _(r):
        pltpu.sync_copy(zi_hbm, hist_v.at[pl.ds(r, _NPT)])
    base_e = wid * _EPT

    def issue(c, buf, sem):
        pltpu.async_copy(col_hbm.at[pl.ds(base_e + c * _CH3, _CH3)], buf, sem)

    def wait(buf, sem):
        pltpu.make_async_copy(col_hbm.at[pl.ds(base_e, _CH3)], buf, sem).wait()

    issue(0, col_v0, sem0)
    @pl.loop(0, _EPT // _CH3 - 1, step=2)
    def _(c):
        issue(c + 1, col_v1, sem1)
        wait(col_v0, sem0)
        _hist_chunk(col_v0, hist_v)
        issue(c + 2, col_v0, sem0)
        wait(col_v1, sem1)
        _hist_chunk(col_v1, hist_v)
    wait(col_v0, sem0)
    _hist_chunk(col_v0, hist_v)
    pltpu.sync_copy(hist_v, hist_hbm.at[wid])


def _sc_hist(col):
    mesh = plsc.VectorSubcoreMesh(core_axis_name="c", subcore_axis_name="s")
    run = pl.kernel(
        _sc_hist_body,
        mesh=mesh,
        compiler_params=_SC_PARAMS_NL,
        out_type=jax.ShapeDtypeStruct((_NTILES, _NPAD), jnp.int32),
        scratch_types=[
            pltpu.VMEM((_NPAD,), jnp.int32),
            pltpu.VMEM((_CH3,), jnp.int32),
            pltpu.VMEM((_CH3,), jnp.int32),
            pltpu.SemaphoreType.DMA,
            pltpu.SemaphoreType.DMA,
        ],
    )
    return run(col, jnp.zeros((_NPT,), jnp.int32))


_BC = 4352


def _offsets_body(h_ref, bases_ref, csr_ref, cnt_ref, carry_ref):
    @pl.when(pl.program_id(0) == 0)
    def _():
        carry_ref[0, 0] = 0
    h = h_ref[...]
    tot = jnp.sum(h, axis=0, keepdims=True)
    s = tot
    sh = 1
    while sh < _BC:
        s = s + jnp.concatenate(
            [jnp.zeros((1, sh), jnp.int32), s[:, :-sh]], axis=1)
        sh *= 2
    ex_n = s - tot + carry_ref[0, 0]
    run = jnp.zeros((1, _BC), jnp.int32)
    rows = []
    for t in range(_NTILES):
        rows.append(run)
        run = run + h[t:t + 1]
    ex_t = jnp.concatenate(rows, axis=0)
    bases_ref[...] = ex_t + ex_n
    csr_ref[...] = ex_n
    cnt_ref[...] = tot
    carry_ref[0, 0] = carry_ref[0, 0] + jnp.sum(h)


def _tc_offsets(hist):
    return pl.pallas_call(
        _offsets_body,
        grid=(_NPAD // _BC,),
        in_specs=[pl.BlockSpec((_NTILES, _BC), lambda j: (0, j))],
        out_specs=[pl.BlockSpec((_NTILES, _BC), lambda j: (0, j)),
                   pl.BlockSpec((1, _BC), lambda j: (0, j)),
                   pl.BlockSpec((1, _BC), lambda j: (0, j))],
        out_shape=(jax.ShapeDtypeStruct((_NTILES, _NPAD), jnp.int32),
                   jax.ShapeDtypeStruct((1, _NPAD), jnp.int32),
                   jax.ShapeDtypeStruct((1, _NPAD), jnp.int32)),
        scratch_shapes=[pltpu.SMEM((1, 1), jnp.int32)],
    )(hist)


def _place_chunk(col_v, attr_v, slots_v, tmp_v, next_v, rows_hbm):
    @pl.loop(0, _CH3, step=16)
    def _(v):
        cv = col_v[pl.ds(v, 16)]
        sk, sv, rank, last = _vec_ranks(cv)
        base = plsc.load_gather(next_v, [sk])
        slot = base + rank
        plsc.store_scatter(next_v, [sk], slot + 1, mask=last)
        plsc.store_scatter(tmp_v, [sv], slot)
        slots_v[pl.ds(v, 16)] = tmp_v[...]
    pltpu.sync_copy(attr_v, rows_hbm.at[slots_v])


def _sc_place_body(col_hbm, attr_hbm, bases_hbm, rows_hbm,
                   next_v, col_v0, attr_v0, slots_v0, col_v1, attr_v1,
                   slots_v1, tmp_v, semc0, sema0, semc1, sema1):
    wid = lax.axis_index("s") * 2 + lax.axis_index("c")
    pltpu.sync_copy(bases_hbm.at[wid], next_v)
    base_e = wid * _EPT

    def issue(c, colv, attrv, semc, sema):
        e0 = base_e + c * _CH3
        pltpu.async_copy(col_hbm.at[pl.ds(e0, _CH3)], colv, semc)
        pltpu.async_copy(attr_hbm.at[pl.ds(e0, _CH3)], attrv, sema)

    def wait(colv, attrv, semc, sema):
        pltpu.make_async_copy(col_hbm.at[pl.ds(base_e, _CH3)], colv,
                              semc).wait()
        pltpu.make_async_copy(attr_hbm.at[pl.ds(base_e, _CH3)], attrv,
                              sema).wait()

    issue(0, col_v0, attr_v0, semc0, sema0)
    @pl.loop(0, _EPT // _CH3 - 1, step=2)
    def _(c):
        issue(c + 1, col_v1, attr_v1, semc1, sema1)
        wait(col_v0, attr_v0, semc0, sema0)
        _place_chunk(col_v0, attr_v0, slots_v0, tmp_v, next_v, rows_hbm)
        issue(c + 2, col_v0, attr_v0, semc0, sema0)
        wait(col_v1, attr_v1, semc1, sema1)
        _place_chunk(col_v1, attr_v1, slots_v1, tmp_v, next_v, rows_hbm)
    wait(col_v0, attr_v0, semc0, sema0)
    _place_chunk(col_v0, attr_v0, slots_v0, tmp_v, next_v, rows_hbm)


def _sc_place(col, attr, bases):
    mesh = plsc.VectorSubcoreMesh(core_axis_name="c", subcore_axis_name="s")
    run = pl.kernel(
        _sc_place_body,
        mesh=mesh,
        compiler_params=_SC_PARAMS_NL,
        out_type=jax.ShapeDtypeStruct((_EPAD, DE), jnp.float32),
        scratch_types=[
            pltpu.VMEM((_NPAD,), jnp.int32),
            pltpu.VMEM((_CH3,), jnp.int32),
            pltpu.VMEM((_CH3, DE), jnp.float32),
            pltpu.VMEM((_CH3,), jnp.int32),
            pltpu.VMEM((_CH3,), jnp.int32),
            pltpu.VMEM((_CH3, DE), jnp.float32),
            pltpu.VMEM((_CH3,), jnp.int32),
            pltpu.VMEM((16,), jnp.int32),
            pltpu.SemaphoreType.DMA,
            pltpu.SemaphoreType.DMA,
            pltpu.SemaphoreType.DMA,
            pltpu.SemaphoreType.DMA,
        ],
    )
    return run(col, attr, bases)


_NG = 196


def _sc_reduce_body(rows_hbm, csr_hbm, cnt_hbm, omax_hbm, osum_hbm, ocnf_hbm,
                    rows_v, stage_m, stage_s, cntc_v, cntf_v, seg_v):
    wid = lax.axis_index("s") * 2 + lax.axis_index("c")
    n0 = wid * _NPT32
    lane = lax.iota(jnp.int32, 16)
    cntc_v[pl.ds(_NPT32 - 8, 16)] = jnp.zeros((16,), jnp.int32)
    pltpu.sync_copy(cnt_hbm.at[0].at[pl.ds(n0, _NPT32)],
                    cntc_v.at[pl.ds(0, _NPT32)])
    pltpu.sync_copy(csr_hbm.at[0].at[pl.ds(n0, 16)], seg_v)
    seg0 = jnp.sum(jnp.where(lane == 0, seg_v[...], 0))
    base_al = seg0 // 8 * 8
    pltpu.sync_copy(rows_hbm.at[pl.ds(base_al, _WIN)], rows_v)
    p0 = seg0 - base_al
    neg = jnp.full((16,), -jnp.inf, jnp.float32)
    zro = jnp.full((16,), 0.0, jnp.float32)

    def group_body(g, carry):
        p, wcur = carry
        cvec = cntc_v[pl.ds(g * 16, 16)]
        cntf_v[pl.ds(g * 16, 16)] = cvec.astype(jnp.float32)

        def node_body(i, nc):
            p, wcur = nc
            k = jnp.sum(jnp.where(lane == i, cvec, 0))
            acc0m = jnp.where(k > 0, neg, zro)

            def run_cond(st):
                krem, _p, _w, _am, _as_ = st
                return krem > 0

            def run_body(st):
                krem, p, wcur, accm, accs = st
                refill = p == _WIN
                wcur = jnp.where(refill, wcur + _WIN, wcur)
                @pl.when(refill)
                def _():
                    pltpu.sync_copy(rows_hbm.at[pl.ds(wcur, _WIN)], rows_v)
                p = jnp.where(refill, 0, p)
                r = jnp.minimum(krem, _WIN - p)

                def pair_body(_j, ec):
                    pe, am, asum = ec
                    r0 = rows_v[pe]
                    r1 = rows_v[pe + 1]
                    return (pe + 2, jnp.maximum(am, jnp.maximum(r0, r1)),
                            asum + r0 + r1)

                def one_body(_j, ec):
                    pe, am, asum = ec
                    r0 = rows_v[pe]
                    return (pe + 1, jnp.maximum(am, r0), asum + r0)

                p, accm, accs = lax.fori_loop(0, r // 2, pair_body,
                                              (p, accm, accs))
                p, accm, accs = lax.fori_loop(0, r % 2, one_body,
                                              (p, accm, accs))
                return (krem - r, p, wcur, accm, accs)

            _k, p, wcur, accm, accs = lax.while_loop(
                run_cond, run_body, (k, p, wcur, acc0m, zro))
            stage_m[g * 16 + i] = accm
            stage_s[g * 16 + i] = accs
            return (p, wcur)

        return lax.fori_loop(0, 16, node_body, (p, wcur))

    lax.fori_loop(0, _NG, group_body, (p0, base_al))
    pltpu.sync_copy(stage_m.at[pl.ds(0, _NPT32)],
                    omax_hbm.at[pl.ds(n0, _NPT32)])
    pltpu.sync_copy(stage_s.at[pl.ds(0, _NPT32)],
                    osum_hbm.at[pl.ds(n0, _NPT32)])
    pltpu.sync_copy(cntf_v.at[pl.ds(0, _NPT32)],
                    ocnf_hbm.at[pl.ds(n0, _NPT32)])


def _sc_reduce(rows, csr, cnt):
    mesh = plsc.VectorSubcoreMesh(core_axis_name="c", subcore_axis_name="s")
    run = pl.kernel(
        _sc_reduce_body,
        mesh=mesh,
        compiler_params=_SC_PARAMS_NL,
        out_type=(jax.ShapeDtypeStruct((_NPAD, DE), jnp.float32),
                  jax.ShapeDtypeStruct((_NPAD, DE), jnp.float32),
                  jax.ShapeDtypeStruct((_NPAD,), jnp.float32)),
        scratch_types=[
            pltpu.VMEM((_WIN, DE), jnp.float32),
            pltpu.VMEM((_NG * 16, DE), jnp.float32),
            pltpu.VMEM((_NG * 16, DE), jnp.float32),
            pltpu.VMEM((_NG * 16,), jnp.int32),
            pltpu.VMEM((_NG * 16,), jnp.float32),
            pltpu.VMEM((16,), jnp.int32),
        ],
    )
    return run(rows, csr, cnt)


_BLK = 2000


def _mlp_body(x_ref, s_ref, c_ref, m_ref, b_ref, u_ref, w1_ref, b1_ref,
              w2_ref, b2_ref, o_ref):
    x = x_ref[...]
    s = s_ref[...]
    c = c_ref[...]
    m = jnp.where(c > 0, m_ref[...], 0.0)
    mn = s / jnp.maximum(c, 1.0)
    oh = (b_ref[...] == jax.lax.broadcasted_iota(jnp.int32, (_BLK, G), 1))
    ub = jnp.dot(oh.astype(jnp.float32), u_ref[...],
                 preferred_element_type=jnp.float32)
    cat = jnp.concatenate([x, s, m, mn, ub], axis=1)
    h = jnp.dot(cat, w1_ref[...], preferred_element_type=jnp.float32)
    h = jnp.maximum(h + b1_ref[...], 0.0)
    o = jnp.dot(h, w2_ref[...], preferred_element_type=jnp.float32)
    o_ref[...] = o + b2_ref[...] + x


def _mlp(x, sum2, cnt2, maxraw, batch2d, u, W1, b1, W2, b2):
    nblk = N // _BLK
    rep = lambda i: (0, 0)
    return pl.pallas_call(
        _mlp_body,
        grid=(nblk,),
        in_specs=[
            pl.BlockSpec((_BLK, D), lambda i: (i, 0)),
            pl.BlockSpec((_BLK, DE), lambda i: (i, 0)),
            pl.BlockSpec((_BLK, 1), lambda i: (i, 0)),
            pl.BlockSpec((_BLK, DE), lambda i: (i, 0)),
            pl.BlockSpec((_BLK, 1), lambda i: (i, 0)),
            pl.BlockSpec((G, 16), rep),
            pl.BlockSpec((IN_DIM, H), rep),
            pl.BlockSpec((1, H), rep),
            pl.BlockSpec((H, D), rep),
            pl.BlockSpec((1, D), rep),
        ],
        out_specs=pl.BlockSpec((_BLK, D), lambda i: (i, 0)),
        out_shape=jax.ShapeDtypeStruct((N, D), jnp.float32),
    )(x, sum2, cnt2, maxraw, batch2d, u, W1, b1, W2, b2)


def kernel(x, edge_index, edge_attr, u, batch, W1, b1, W2, b2):
    col = edge_index[1]
    hist = _sc_hist(col)
    bases, csr, cnt = _tc_offsets(hist)
    rows = _sc_place(col, edge_attr, bases)
    maxraw, sumv, cnf = _sc_reduce(rows, csr, cnt)
    return _mlp(x, sumv, cnf.reshape(_NPAD, 1), maxraw,
                batch.reshape(N, 1), u, W1, b1.reshape(1, H), W2,
                b2.reshape(1, D))
